# Initial kernel scaffold; baseline (speedup 1.0000x reference)
import jax
import jax.numpy as jnp
from jax.experimental import pallas as pl


def kernel(ori_pcs, adv_pcs, ori_normals):
    raise NotImplementedError("write your pallas kernel here")


# speedup vs baseline: 21.2543x; 21.2543x over previous
import jax
import jax.numpy as jnp
from jax.experimental import pallas as pl
from jax.experimental.pallas import tpu as pltpu

_B, _N = 8, 2048
_BR = 256
_NBLK = _N // _BR
_EPS = 1e-12


def _bf(x):
    return x.astype(jnp.bfloat16).astype(jnp.float32)


def _sel_dist(rb, cb):
    bb = rb[0] * rb[0] + rb[1] * rb[1] + rb[2] * rb[2]
    aa = cb[0] * cb[0] + cb[1] * cb[1] + cb[2] * cb[2]
    ab = (_bf(cb[0]) * _bf(rb[0]) + _bf(cb[1]) * _bf(rb[1])
          + _bf(cb[2]) * _bf(rb[2]))
    return (aa + bb) - 2.0 * ab


def _argmin_mask(dmat, iota):
    dmin = jnp.min(dmat, axis=1, keepdims=True)
    jmin = jnp.min(jnp.where(dmat == dmin, iota, _N), axis=1, keepdims=True)
    return iota == jmin


def _min3_masks(dmat, iota):
    inf = jnp.float32(jnp.inf)
    m1 = _argmin_mask(dmat, iota)
    dm = jnp.where(m1, inf, dmat)
    m2 = _argmin_mask(dm, iota)
    dm2 = jnp.where(m2, inf, dm)
    m3 = _argmin_mask(dm2, iota)
    return m2, m3


def _extract(mask, rows):
    return [jnp.sum(jnp.where(mask, rows[d], 0.0), axis=1, keepdims=True)
            for d in range(3)]


def _contrib(mask, rb, cb, nv):
    px, py, pz = _extract(mask, rb)
    dx = px - cb[0]
    dy = py - cb[1]
    dz = pz - cb[2]
    norm = jnp.sqrt(dx * dx + dy * dy + dz * dz)
    inv = 1.0 / jnp.maximum(norm, _EPS)
    return jnp.abs((dx * inv) * nv[0] + (dy * inv) * nv[1] + (dz * inv) * nv[2])


def _cloud_kappa(rb, cb, nv, iota):
    dsel = _sel_dist(rb, cb)
    m2, m3 = _min3_masks(dsel, iota)
    return (_contrib(m2, rb, cb, nv) + _contrib(m3, rb, cb, nv)) * 0.5


def _body(ot_ref, at_ref, nt_ref, o_ref, a_ref, n_ref, out_ref, acc_ref):
    b = pl.program_id(0)
    r = pl.program_id(1)
    pt = ot_ref[0]
    at = at_ref[0]
    nt = nt_ref[0]
    ob = o_ref[0]
    ab_ = a_ref[0]
    nb = n_ref[0]
    iota = jax.lax.broadcasted_iota(jnp.int32, (_BR, _N), 1)

    @pl.when(jnp.logical_and(b == 0, r == 0))
    def _():
        acc_ref[4] = 0.0

    @pl.when(r == 0)
    def _():
        acc_ref[0] = 0.0
        acc_ref[1] = 0.0
        acc_ref[2] = 0.0
        acc_ref[3] = 0.0

    prow = [pt[d:d + 1, :] for d in range(3)]
    arow = [at[d:d + 1, :] for d in range(3)]
    nrow = [nt[d:d + 1, :] for d in range(3)]
    ocol = [ob[:, d:d + 1] for d in range(3)]
    acol = [ab_[:, d:d + 1] for d in range(3)]
    ncol = [nb[:, d:d + 1] for d in range(3)]

    ko = _cloud_kappa(prow, ocol, ncol, iota)

    dao = _sel_dist(prow, acol)
    m1 = _argmin_mask(dao, iota)
    nh = _extract(m1, nrow)

    ka = _cloud_kappa(arow, acol, nh, iota)

    acc_ref[0] += jnp.sum(ko)
    acc_ref[1] += jnp.sum(ko * ko)
    acc_ref[2] += jnp.sum(ka)
    acc_ref[3] += jnp.sum(ka * ka)

    @pl.when(r == _NBLK - 1)
    def _():
        n = jnp.float32(_N)
        var_o = (acc_ref[1] - acc_ref[0] * acc_ref[0] / n) / (n - 1.0)
        var_a = (acc_ref[3] - acc_ref[2] * acc_ref[2] / n) / (n - 1.0)
        std_o = jnp.sqrt(jnp.maximum(var_o, 0.0))
        std_a = jnp.sqrt(jnp.maximum(var_a, 0.0))
        acc_ref[4] += jnp.abs(std_a - std_o)

    @pl.when(jnp.logical_and(b == _B - 1, r == _NBLK - 1))
    def _():
        out_ref[...] = jnp.full((1, 1), acc_ref[4] / jnp.float32(_B), jnp.float32)


def _call(ori_pcs, adv_pcs, ori_normals, interpret=False):
    ori_t = ori_pcs.transpose(0, 2, 1)
    adv_t = adv_pcs.transpose(0, 2, 1)
    nrm_t = ori_normals.transpose(0, 2, 1)
    out = pl.pallas_call(
        _body,
        grid=(_B, _NBLK),
        in_specs=[
            pl.BlockSpec((1, 3, _N), lambda b, r: (b, 0, 0)),
            pl.BlockSpec((1, 3, _N), lambda b, r: (b, 0, 0)),
            pl.BlockSpec((1, 3, _N), lambda b, r: (b, 0, 0)),
            pl.BlockSpec((1, _BR, 3), lambda b, r: (b, r, 0)),
            pl.BlockSpec((1, _BR, 3), lambda b, r: (b, r, 0)),
            pl.BlockSpec((1, _BR, 3), lambda b, r: (b, r, 0)),
        ],
        out_specs=pl.BlockSpec((1, 1), lambda b, r: (0, 0)),
        out_shape=jax.ShapeDtypeStruct((1, 1), jnp.float32),
        scratch_shapes=[pltpu.SMEM((8,), jnp.float32)],
        interpret=interpret,
    )(ori_t, adv_t, nrm_t, ori_pcs, adv_pcs, ori_normals)
    return out[0, 0]


def kernel(ori_pcs, adv_pcs, ori_normals):
    return _call(ori_pcs, adv_pcs, ori_normals)
